# Initial kernel scaffold; baseline (speedup 1.0000x reference)
import functools

import jax
import jax.numpy as jnp
from jax import lax
from jax.experimental import pallas as pl
from jax.experimental.pallas import tpu as pltpu
from jax.experimental.pallas import tpu_sc as plsc

_NC = 2
_NS = 16
_NW = _NC * _NS
_L = 16

_NB = 512


def _tc_argmin_body(u_ref, k_ref, o_ref):
    acc = None
    for c in range(3):
        uc = u_ref[0, c, :]
        kc = k_ref[0, c, :]
        d = uc[:, None] - kc[None, :]
        sq = d * d
        acc = sq if acc is None else acc + sq
    o_ref[0, 0, :] = jnp.argmin(acc, axis=1).astype(jnp.int32)


def _tc_argmin(u_t, k_t):
    B, _, N = u_t.shape
    M = k_t.shape[2]
    nblk = N // _NB
    grid = (B * nblk,)
    idx3 = pl.pallas_call(
        _tc_argmin_body,
        grid=grid,
        in_specs=[
            pl.BlockSpec((1, 3, _NB), lambda i: (i // nblk, 0, i % nblk)),
            pl.BlockSpec((1, 3, M), lambda i: (i // nblk, 0, 0)),
        ],
        out_specs=pl.BlockSpec((1, 1, _NB), lambda i: (i, 0, 0)),
        out_shape=jax.ShapeDtypeStruct((B * nblk, 1, _NB), jnp.int32),
    )(u_t, k_t)
    return idx3.reshape(B, N)


def _sc_gather(known_feats, idx):
    B, C, M = known_feats.shape
    N = idx.shape[1]
    rows_per_tile = (B * C) // _NW
    tiles_per_b = _NW // B
    nvec = N // _L

    mesh = plsc.VectorSubcoreMesh(core_axis_name="c", subcore_axis_name="s")

    @functools.partial(
        pl.kernel,
        out_type=jax.ShapeDtypeStruct((B, C, N), jnp.float32),
        mesh=mesh,
        scratch_types=[
            pltpu.VMEM((N,), jnp.int32),
            pltpu.VMEM((M,), jnp.float32),
            pltpu.VMEM((N,), jnp.float32),
        ],
    )
    def k(feats_hbm, idx_hbm, out_hbm, idx_v, row_v, out_v):
        cid = lax.axis_index("c")
        sid = lax.axis_index("s")
        wid = sid * _NC + cid
        b = wid // tiles_per_b
        c0 = (wid % tiles_per_b) * rows_per_tile
        pltpu.sync_copy(idx_hbm.at[b], idx_v)

        def row_body(r, _):
            pltpu.sync_copy(feats_hbm.at[b, c0 + r], row_v)

            def gather_body(j, _):
                iv = idx_v[pl.ds(j * _L, _L)]
                out_v[pl.ds(j * _L, _L)] = plsc.load_gather(row_v, [iv])
                return 0

            lax.fori_loop(0, nvec, gather_body, 0)
            pltpu.sync_copy(out_v, out_hbm.at[b, c0 + r])
            return 0

        lax.fori_loop(0, rows_per_tile, row_body, 0)

    return k(known_feats, idx)


def kernel(unknown, known, known_feats):
    u_t = jnp.transpose(unknown, (0, 2, 1))
    k_t = jnp.transpose(known, (0, 2, 1))
    idx = _tc_argmin(u_t, k_t)
    return _sc_gather(known_feats, idx)


# speedup vs baseline: 1062.4476x; 1062.4476x over previous
import functools

import jax
import jax.numpy as jnp
from jax import lax
from jax.experimental import pallas as pl
from jax.experimental.pallas import tpu as pltpu
from jax.experimental.pallas import tpu_sc as plsc

_NC = 2
_NS = 16
_NW = _NC * _NS
_L = 16

_NB = 512


def _tc_argmin_body(u_ref, k_ref, o_ref):
    acc = None
    for c in range(3):
        uc = u_ref[0, c, :]
        kc = k_ref[0, c, :]
        d = uc[:, None] - kc[None, :]
        sq = d * d
        acc = sq if acc is None else acc + sq
    o_ref[0, 0, :] = jnp.argmin(acc, axis=1).astype(jnp.int32)


def _tc_argmin(u_t, k_t):
    B, _, N = u_t.shape
    M = k_t.shape[2]
    nblk = N // _NB
    grid = (B * nblk,)
    idx3 = pl.pallas_call(
        _tc_argmin_body,
        grid=grid,
        in_specs=[
            pl.BlockSpec((1, 3, _NB), lambda i: (i // nblk, 0, i % nblk)),
            pl.BlockSpec((1, 3, M), lambda i: (i // nblk, 0, 0)),
        ],
        out_specs=pl.BlockSpec((1, 1, _NB), lambda i: (i, 0, 0)),
        out_shape=jax.ShapeDtypeStruct((B * nblk, 1, _NB), jnp.int32),
    )(u_t, k_t)
    return idx3.reshape(B, N)


def _sc_gather(known_feats, idx):
    B, C, M = known_feats.shape
    N = idx.shape[1]
    rows_per_tile = (B * C) // _NW
    tiles_per_b = _NW // B
    nvec = N // _L

    mesh = plsc.VectorSubcoreMesh(core_axis_name="c", subcore_axis_name="s")

    @functools.partial(
        pl.kernel,
        out_type=jax.ShapeDtypeStruct((B, C, N), jnp.float32),
        mesh=mesh,
        compiler_params=pltpu.CompilerParams(needs_layout_passes=False),
        scratch_types=[
            pltpu.VMEM((N,), jnp.int32),
            pltpu.VMEM((M,), jnp.float32),
            pltpu.VMEM((N,), jnp.float32),
        ],
    )
    def k(feats_hbm, idx_hbm, out_hbm, idx_v, row_v, out_v):
        cid = lax.axis_index("c")
        sid = lax.axis_index("s")
        wid = sid * _NC + cid
        b = wid // tiles_per_b
        c0 = (wid % tiles_per_b) * rows_per_tile
        pltpu.sync_copy(idx_hbm.at[b], idx_v)

        def row_body(r, _):
            pltpu.sync_copy(feats_hbm.at[b, c0 + r], row_v)

            def gather_body(j, _):
                iv = idx_v[pl.ds(j * _L, _L)]
                out_v[pl.ds(j * _L, _L)] = plsc.load_gather(row_v, [iv])
                return 0

            lax.fori_loop(0, nvec, gather_body, 0)
            pltpu.sync_copy(out_v, out_hbm.at[b, c0 + r])
            return 0

        lax.fori_loop(0, rows_per_tile, row_body, 0)

    return k(known_feats, idx)


def kernel(unknown, known, known_feats):
    u_t = jnp.transpose(unknown, (0, 2, 1))
    k_t = jnp.transpose(known, (0, 2, 1))
    idx = _tc_argmin(u_t, k_t)
    return _sc_gather(known_feats, idx)


# speedup vs baseline: 2471.3939x; 2.3261x over previous
import functools

import jax
import jax.numpy as jnp
from jax import lax
from jax.experimental import pallas as pl
from jax.experimental.pallas import tpu as pltpu
from jax.experimental.pallas import tpu_sc as plsc

_NC = 2
_NS = 16
_NW = _NC * _NS
_L = 16

_NB = 512


def _tc_argmin_body(u_ref, k_ref, o_ref):
    acc = None
    for c in range(3):
        uc = u_ref[0, c, :]
        kc = k_ref[0, c, :]
        d = uc[:, None] - kc[None, :]
        sq = d * d
        acc = sq if acc is None else acc + sq
    o_ref[0, 0, :] = jnp.argmin(acc, axis=1).astype(jnp.int32)


def _tc_argmin(u_t, k_t):
    B, _, N = u_t.shape
    M = k_t.shape[2]
    nblk = N // _NB
    grid = (B * nblk,)
    idx3 = pl.pallas_call(
        _tc_argmin_body,
        grid=grid,
        in_specs=[
            pl.BlockSpec((1, 3, _NB), lambda i: (i // nblk, 0, i % nblk)),
            pl.BlockSpec((1, 3, M), lambda i: (i // nblk, 0, 0)),
        ],
        out_specs=pl.BlockSpec((1, 1, _NB), lambda i: (i, 0, 0)),
        out_shape=jax.ShapeDtypeStruct((B * nblk, 1, _NB), jnp.int32),
    )(u_t, k_t)
    return idx3.reshape(B, N)


def _sc_gather(known_feats, idx):
    B, C, M = known_feats.shape
    N = idx.shape[1]
    rows_per_tile = (B * C) // _NW
    tiles_per_b = _NW // B
    R = 8
    n_groups = rows_per_tile // R
    NCH = 4096
    n_chunks = N // NCH
    nv = NCH // _L

    mesh = plsc.VectorSubcoreMesh(core_axis_name="c", subcore_axis_name="s")

    @functools.partial(
        pl.kernel,
        out_type=jax.ShapeDtypeStruct((B, C, N), jnp.float32),
        mesh=mesh,
        compiler_params=pltpu.CompilerParams(needs_layout_passes=False),
        scratch_types=[
            pltpu.VMEM((N,), jnp.int32),
        ]
        + [pltpu.VMEM((M,), jnp.float32) for _ in range(R)]
        + [
            pltpu.VMEM((2, R, NCH), jnp.float32),
            pltpu.SemaphoreType.DMA,
            pltpu.SemaphoreType.DMA,
        ],
    )
    def k(feats_hbm, idx_hbm, out_hbm, idx_v, *rest):
        feats_rows = rest[:R]
        out_v, osem0, osem1 = rest[R:]
        cid = lax.axis_index("c")
        sid = lax.axis_index("s")
        wid = sid * _NC + cid
        b = wid // tiles_per_b
        c0 = (wid % tiles_per_b) * rows_per_tile
        pltpu.sync_copy(idx_hbm.at[b], idx_v)

        osems = (osem0, osem1)
        pending = [None, None]
        kflat = 0
        for g in range(n_groups):
            for r in range(R):
                pltpu.sync_copy(feats_hbm.at[b, c0 + g * R + r], feats_rows[r])
            for t in range(n_chunks):
                bsel = kflat % 2
                if pending[bsel] is not None:
                    pending[bsel].wait()
                ob = out_v.at[bsel]

                @plsc.parallel_loop(0, nv, step=1, unroll=4)
                def vbody(j):
                    iv = idx_v[pl.ds(t * NCH + j * _L, _L)]
                    for r in range(R):
                        ob[r, pl.ds(j * _L, _L)] = plsc.load_gather(
                            feats_rows[r], [iv])

                dst = out_hbm.at[b, pl.ds(c0 + g * R, R), pl.ds(t * NCH, NCH)]
                pending[bsel] = pltpu.async_copy(ob, dst, osems[bsel])
                kflat += 1
        for p in pending:
            if p is not None:
                p.wait()

    return k(known_feats, idx)


def kernel(unknown, known, known_feats):
    u_t = jnp.transpose(unknown, (0, 2, 1))
    k_t = jnp.transpose(known, (0, 2, 1))
    idx = _tc_argmin(u_t, k_t)
    return _sc_gather(known_feats, idx)


# speedup vs baseline: 3160.9329x; 1.2790x over previous
import functools

import jax
import jax.numpy as jnp
from jax import lax
from jax.experimental import pallas as pl
from jax.experimental.pallas import tpu as pltpu
from jax.experimental.pallas import tpu_sc as plsc

_NC = 2
_NS = 16
_NW = _NC * _NS
_L = 16

_NB = 512


def _tc_argmin_body(u_ref, k_ref, o_ref):
    acc = None
    for c in range(3):
        uc = u_ref[0, c, :]
        kc = k_ref[0, c, :]
        d = uc[:, None] - kc[None, :]
        sq = d * d
        acc = sq if acc is None else acc + sq
    o_ref[0, 0, :] = jnp.argmin(acc, axis=1).astype(jnp.int32)


def _tc_argmin(u_t, k_t):
    B, _, N = u_t.shape
    M = k_t.shape[2]
    nblk = N // _NB
    grid = (B * nblk,)
    idx3 = pl.pallas_call(
        _tc_argmin_body,
        grid=grid,
        in_specs=[
            pl.BlockSpec((1, 3, _NB), lambda i: (i // nblk, 0, i % nblk)),
            pl.BlockSpec((1, 3, M), lambda i: (i // nblk, 0, 0)),
        ],
        out_specs=pl.BlockSpec((1, 1, _NB), lambda i: (i, 0, 0)),
        out_shape=jax.ShapeDtypeStruct((B * nblk, 1, _NB), jnp.int32),
    )(u_t, k_t)
    return idx3.reshape(B, N)


def _sc_gather(known_feats, idx):
    B, C, M = known_feats.shape
    N = idx.shape[1]
    rows_per_tile = (B * C) // _NW
    tiles_per_b = _NW // B
    R = 8
    n_groups = rows_per_tile // R
    NCH = 4096
    n_chunks = N // NCH
    nv = NCH // _L

    mesh = plsc.VectorSubcoreMesh(core_axis_name="c", subcore_axis_name="s")

    @functools.partial(
        pl.kernel,
        out_type=jax.ShapeDtypeStruct((B, C, N), jnp.float32),
        mesh=mesh,
        compiler_params=pltpu.CompilerParams(needs_layout_passes=False),
        scratch_types=[
            pltpu.VMEM((N,), jnp.int32),
        ]
        + [pltpu.VMEM((M,), jnp.float32) for _ in range(R)]
        + [
            pltpu.VMEM((2, R, NCH), jnp.float32),
            pltpu.SemaphoreType.DMA,
            pltpu.SemaphoreType.DMA,
        ],
    )
    def k(feats_hbm, idx_hbm, out_hbm, idx_v, *rest):
        feats_rows = rest[:R]
        out_v, osem0, osem1 = rest[R:]
        cid = lax.axis_index("c")
        sid = lax.axis_index("s")
        wid = sid * _NC + cid
        b = wid // tiles_per_b
        c0 = (wid % tiles_per_b) * rows_per_tile
        pltpu.sync_copy(idx_hbm.at[b], idx_v)

        osems = (osem0, osem1)
        pending = [None, None]
        kflat = 0
        for g in range(n_groups):
            for r in range(R):
                pltpu.sync_copy(feats_hbm.at[b, c0 + g * R + r], feats_rows[r])
            for t in range(n_chunks):
                bsel = kflat % 2
                if pending[bsel] is not None:
                    pending[bsel].wait()
                ob = out_v.at[bsel]

                @plsc.parallel_loop(0, nv, step=1, unroll=4)
                def vbody(j):
                    iv = idx_v[pl.ds(t * NCH + j * _L, _L)]
                    for r in range(R):
                        ob[r, pl.ds(j * _L, _L)] = plsc.load_gather(
                            feats_rows[r], [iv])

                dst = out_hbm.at[b, pl.ds(c0 + g * R, R), pl.ds(t * NCH, NCH)]
                pending[bsel] = pltpu.async_copy(ob, dst, osems[bsel])
                kflat += 1
        for p in pending:
            if p is not None:
                p.wait()

    return k(known_feats, idx)


def _tc_argmin_body_b(u_ref, k_ref, o_ref):
    acc = None
    for c in range(3):
        d = u_ref[c, :][:, None] - k_ref[c, :][None, :]
        sq = d * d
        acc = sq if acc is None else acc + sq
    o_ref[0, 0, :] = jnp.argmin(acc, axis=1).astype(jnp.int32)


def _tc_argmin_b(u_t, k_t):
    _, N = u_t.shape
    M = k_t.shape[1]
    nblk = N // _NB
    idx3 = pl.pallas_call(
        _tc_argmin_body_b,
        grid=(nblk,),
        in_specs=[
            pl.BlockSpec((3, _NB), lambda i: (0, i)),
            pl.BlockSpec((3, M), lambda i: (0, 0)),
        ],
        out_specs=pl.BlockSpec((1, 1, _NB), lambda i: (i, 0, 0)),
        out_shape=jax.ShapeDtypeStruct((nblk, 1, _NB), jnp.int32),
    )(u_t, k_t)
    return idx3.reshape(N)


def _sc_gather_slab(feats_b, idx_b, out_ref, b):
    C, M = feats_b.shape
    N = idx_b.shape[0]
    rows_per_tile = C // _NW
    R = 8
    n_groups = rows_per_tile // R
    NCH = 4096
    n_chunks = N // NCH
    nv = NCH // _L

    mesh = plsc.VectorSubcoreMesh(core_axis_name="c", subcore_axis_name="s")

    @functools.partial(
        pl.kernel,
        out_type=(),
        mesh=mesh,
        compiler_params=pltpu.CompilerParams(needs_layout_passes=False),
        scratch_types=[
            pltpu.VMEM((N,), jnp.int32),
        ]
        + [pltpu.VMEM((M,), jnp.float32) for _ in range(R)]
        + [
            pltpu.VMEM((2, R, NCH), jnp.float32),
            pltpu.SemaphoreType.DMA,
            pltpu.SemaphoreType.DMA,
        ],
    )
    def k(feats_hbm, idx_hbm, out_hbm, idx_v, *rest):
        feats_rows = rest[:R]
        out_v, osem0, osem1 = rest[R:]
        cid = lax.axis_index("c")
        sid = lax.axis_index("s")
        wid = sid * _NC + cid
        c0 = wid * rows_per_tile
        pltpu.sync_copy(idx_hbm, idx_v)

        osems = (osem0, osem1)
        pending = [None, None]
        kflat = 0
        for g in range(n_groups):
            for r in range(R):
                pltpu.sync_copy(feats_hbm.at[c0 + g * R + r], feats_rows[r])
            for t in range(n_chunks):
                bsel = kflat % 2
                if pending[bsel] is not None:
                    pending[bsel].wait()
                ob = out_v.at[bsel]

                @plsc.parallel_loop(0, nv, step=1, unroll=4)
                def vbody(j):
                    iv = idx_v[pl.ds(t * NCH + j * _L, _L)]
                    for r in range(R):
                        ob[r, pl.ds(j * _L, _L)] = plsc.load_gather(
                            feats_rows[r], [iv])

                dst = out_hbm.at[b, pl.ds(c0 + g * R, R), pl.ds(t * NCH, NCH)]
                pending[bsel] = pltpu.async_copy(ob, dst, osems[bsel])
                kflat += 1
        for p in pending:
            if p is not None:
                p.wait()

    k(feats_b, idx_b, out_ref)


def kernel(unknown, known, known_feats):
    B, N, _ = unknown.shape
    C = known_feats.shape[1]
    out_ref = jax.new_ref(jax.lax.empty((B, C, N), jnp.float32))
    for b in range(B):
        u_t = jnp.transpose(unknown[b])
        k_t = jnp.transpose(known[b])
        idx_b = _tc_argmin_b(u_t, k_t)
        _sc_gather_slab(known_feats[b], idx_b, out_ref, b)
    return out_ref[...]


# speedup vs baseline: 3318.8829x; 1.0500x over previous
import functools

import jax
import jax.numpy as jnp
from jax import lax
from jax.experimental import pallas as pl
from jax.experimental.pallas import tpu as pltpu
from jax.experimental.pallas import tpu_sc as plsc

_NC = 2
_NS = 16
_NW = _NC * _NS
_L = 16

_NB = 512


def _tc_argmin_body(u_ref, k_ref, o_ref):
    acc = None
    for c in range(3):
        uc = u_ref[0, c, :]
        kc = k_ref[0, c, :]
        d = uc[:, None] - kc[None, :]
        sq = d * d
        acc = sq if acc is None else acc + sq
    o_ref[0, 0, :] = jnp.argmin(acc, axis=1).astype(jnp.int32)


def _tc_argmin_b(u_t, k_t, b):
    B, _, N = u_t.shape
    M = k_t.shape[2]
    nblk = N // _NB
    idx3 = pl.pallas_call(
        _tc_argmin_body,
        grid=(nblk,),
        in_specs=[
            pl.BlockSpec((1, 3, _NB), lambda i: (b, 0, i)),
            pl.BlockSpec((1, 3, M), lambda i: (b, 0, 0)),
        ],
        out_specs=pl.BlockSpec((1, 1, _NB), lambda i: (i, 0, 0)),
        out_shape=jax.ShapeDtypeStruct((nblk, 1, _NB), jnp.int32),
    )(u_t, k_t)
    return idx3.reshape(N)


def _sc_gather_slab(known_feats, idx_b, out_ref, b):
    B, C, M = known_feats.shape
    N = idx_b.shape[0]
    R = C // _NW
    NCH = 2048
    n_chunks = N // NCH
    nv = NCH // _L

    mesh = plsc.VectorSubcoreMesh(core_axis_name="c", subcore_axis_name="s")

    @functools.partial(
        pl.kernel,
        out_type=(),
        mesh=mesh,
        compiler_params=pltpu.CompilerParams(needs_layout_passes=False),
        scratch_types=[
            pltpu.VMEM((N,), jnp.int32),
        ]
        + [pltpu.VMEM((M,), jnp.float32) for _ in range(R)]
        + [
            pltpu.VMEM((2, R, NCH), jnp.float32),
            pltpu.SemaphoreType.DMA,
            pltpu.SemaphoreType.DMA,
        ],
    )
    def k(feats_hbm, idx_hbm, out_hbm, idx_v, *rest):
        feats_rows = rest[:R]
        out_v, osem0, osem1 = rest[R:]
        cid = lax.axis_index("c")
        sid = lax.axis_index("s")
        wid = sid * _NC + cid
        c0 = wid * R
        pltpu.sync_copy(idx_hbm, idx_v)
        for r in range(R):
            pltpu.sync_copy(feats_hbm.at[b, c0 + r], feats_rows[r])

        osems = (osem0, osem1)
        pending = [None, None]
        for t in range(n_chunks):
            bsel = t % 2
            if pending[bsel] is not None:
                pending[bsel].wait()
            ob = out_v.at[bsel]

            @plsc.parallel_loop(0, nv, step=1, unroll=8)
            def vbody(j):
                iv = idx_v[pl.ds(t * NCH + j * _L, _L)]
                for r in range(R):
                    ob[r, pl.ds(j * _L, _L)] = plsc.load_gather(
                        feats_rows[r], [iv])

            dst = out_hbm.at[b, pl.ds(c0, R), pl.ds(t * NCH, NCH)]
            pending[bsel] = pltpu.async_copy(ob, dst, osems[bsel])
        for p in pending:
            if p is not None:
                p.wait()

    k(known_feats, idx_b, out_ref)


def kernel(unknown, known, known_feats):
    B, N, _ = unknown.shape
    C = known_feats.shape[1]
    u_t = jnp.transpose(unknown, (0, 2, 1))
    k_t = jnp.transpose(known, (0, 2, 1))
    out_ref = jax.new_ref(jax.lax.empty((B, C, N), jnp.float32))
    for b in range(B):
        idx_b = _tc_argmin_b(u_t, k_t, b)
        _sc_gather_slab(known_feats, idx_b, out_ref, b)
    return out_ref[...]


# speedup vs baseline: 3448.9894x; 1.0392x over previous
import functools

import jax
import jax.numpy as jnp
from jax import lax
from jax.experimental import pallas as pl
from jax.experimental.pallas import tpu as pltpu
from jax.experimental.pallas import tpu_sc as plsc

_NC = 2
_NS = 16
_NW = _NC * _NS
_L = 16

_NB = 512


def _tc_argmin_body(u_ref, k_ref, o_ref):
    acc = None
    for c in range(3):
        uc = u_ref[0, c, :]
        kc = k_ref[0, c, :]
        d = uc[:, None] - kc[None, :]
        sq = d * d
        acc = sq if acc is None else acc + sq
    o_ref[0, 0, :] = jnp.argmin(acc, axis=1).astype(jnp.int32)


def _tc_argmin_b(u_t, k_t, b):
    B, _, N = u_t.shape
    M = k_t.shape[2]
    nblk = N // _NB
    idx3 = pl.pallas_call(
        _tc_argmin_body,
        grid=(nblk,),
        in_specs=[
            pl.BlockSpec((1, 3, _NB), lambda i: (b, 0, i)),
            pl.BlockSpec((1, 3, M), lambda i: (b, 0, 0)),
        ],
        out_specs=pl.BlockSpec((1, 1, _NB), lambda i: (i, 0, 0)),
        out_shape=jax.ShapeDtypeStruct((nblk, 1, _NB), jnp.int32),
    )(u_t, k_t)
    return idx3.reshape(N)


def _sc_gather_slab(known_feats, idx_b, out_ref, b):
    B, C, M = known_feats.shape
    N = idx_b.shape[0]
    R = C // _NW
    NCH = 2048
    n_chunks = N // NCH
    nv = NCH // _L

    mesh = plsc.VectorSubcoreMesh(core_axis_name="c", subcore_axis_name="s")

    @functools.partial(
        pl.kernel,
        out_type=(),
        mesh=mesh,
        compiler_params=pltpu.CompilerParams(needs_layout_passes=False),
        scratch_types=[
            pltpu.VMEM((N,), jnp.int32),
        ]
        + [pltpu.VMEM((M,), jnp.float32) for _ in range(R)]
        + [
            pltpu.VMEM((2, R, NCH), jnp.float32),
            pltpu.SemaphoreType.DMA,
            pltpu.SemaphoreType.DMA,
            pltpu.SemaphoreType.DMA,
        ],
    )
    def k(feats_hbm, idx_hbm, out_hbm, idx_v, *rest):
        feats_rows = rest[:R]
        out_v, osem0, osem1, ssem = rest[R:]
        cid = lax.axis_index("c")
        sid = lax.axis_index("s")
        wid = sid * _NC + cid
        c0 = wid * R
        stage = [pltpu.async_copy(idx_hbm, idx_v, ssem)]
        for r in range(R):
            stage.append(
                pltpu.async_copy(feats_hbm.at[b, c0 + r], feats_rows[r], ssem))
        for h in stage:
            h.wait()

        osems = (osem0, osem1)
        pending = [None, None]
        for t in range(n_chunks):
            bsel = t % 2
            if pending[bsel] is not None:
                pending[bsel].wait()
            ob = out_v.at[bsel]

            @plsc.parallel_loop(0, nv, step=1, unroll=8)
            def vbody(j):
                iv = idx_v[pl.ds(t * NCH + j * _L, _L)]
                for r in range(R):
                    ob[r, pl.ds(j * _L, _L)] = plsc.load_gather(
                        feats_rows[r], [iv])

            dst = out_hbm.at[b, pl.ds(c0, R), pl.ds(t * NCH, NCH)]
            pending[bsel] = pltpu.async_copy(ob, dst, osems[bsel])
        for p in pending:
            if p is not None:
                p.wait()

    k(known_feats, idx_b, out_ref)


def kernel(unknown, known, known_feats):
    B, N, _ = unknown.shape
    C = known_feats.shape[1]
    u_t = jnp.transpose(unknown, (0, 2, 1))
    k_t = jnp.transpose(known, (0, 2, 1))
    out_ref = jax.new_ref(jax.lax.empty((B, C, N), jnp.float32))
    for b in range(B):
        idx_b = _tc_argmin_b(u_t, k_t, b)
        _sc_gather_slab(known_feats, idx_b, out_ref, b)
    return out_ref[...]


# speedup vs baseline: 3634.1535x; 1.0537x over previous
import functools

import jax
import jax.numpy as jnp
from jax import lax
from jax.experimental import pallas as pl
from jax.experimental.pallas import tpu as pltpu
from jax.experimental.pallas import tpu_sc as plsc

_NC = 2
_NS = 16
_NW = _NC * _NS
_L = 16

_NB = 2048


def _tc_argmin_body(u_ref, k_ref, o_ref):
    acc = None
    for c in range(3):
        uc = u_ref[0, c, :]
        kc = k_ref[0, c, :]
        d = uc[:, None] - kc[None, :]
        sq = d * d
        acc = sq if acc is None else acc + sq
    o_ref[0, 0, :] = jnp.argmin(acc, axis=1).astype(jnp.int32)


def _tc_argmin_b(u_t, k_t, b):
    B, _, N = u_t.shape
    M = k_t.shape[2]
    nblk = N // _NB
    idx3 = pl.pallas_call(
        _tc_argmin_body,
        grid=(nblk,),
        in_specs=[
            pl.BlockSpec((1, 3, _NB), lambda i: (b, 0, i)),
            pl.BlockSpec((1, 3, M), lambda i: (b, 0, 0)),
        ],
        out_specs=pl.BlockSpec((1, 1, _NB), lambda i: (i, 0, 0)),
        out_shape=jax.ShapeDtypeStruct((nblk, 1, _NB), jnp.int32),
    )(u_t, k_t)
    return idx3.reshape(N)


def _sc_gather_slab(known_feats, idx_b, out_ref, b):
    B, C, M = known_feats.shape
    N = idx_b.shape[0]
    R = C // _NW
    NCH = 2048
    n_chunks = N // NCH
    nv = NCH // _L

    mesh = plsc.VectorSubcoreMesh(core_axis_name="c", subcore_axis_name="s")

    @functools.partial(
        pl.kernel,
        out_type=(),
        mesh=mesh,
        compiler_params=pltpu.CompilerParams(needs_layout_passes=False),
        scratch_types=[
            pltpu.VMEM((N,), jnp.int32),
        ]
        + [pltpu.VMEM((M,), jnp.float32) for _ in range(R)]
        + [
            pltpu.VMEM((2, R, NCH), jnp.float32),
            pltpu.SemaphoreType.DMA,
            pltpu.SemaphoreType.DMA,
            pltpu.SemaphoreType.DMA,
        ],
    )
    def k(feats_hbm, idx_hbm, out_hbm, idx_v, *rest):
        feats_rows = rest[:R]
        out_v, osem0, osem1, ssem = rest[R:]
        cid = lax.axis_index("c")
        sid = lax.axis_index("s")
        wid = sid * _NC + cid
        c0 = wid * R
        stage = [pltpu.async_copy(idx_hbm, idx_v, ssem)]
        for r in range(R):
            stage.append(
                pltpu.async_copy(feats_hbm.at[b, c0 + r], feats_rows[r], ssem))
        for h in stage:
            h.wait()

        osems = (osem0, osem1)
        pending = [None, None]
        for t in range(n_chunks):
            bsel = t % 2
            if pending[bsel] is not None:
                pending[bsel].wait()
            ob = out_v.at[bsel]

            @plsc.parallel_loop(0, nv, step=1, unroll=8)
            def vbody(j):
                iv = idx_v[pl.ds(t * NCH + j * _L, _L)]
                for r in range(R):
                    ob[r, pl.ds(j * _L, _L)] = plsc.load_gather(
                        feats_rows[r], [iv])

            dst = out_hbm.at[b, pl.ds(c0, R), pl.ds(t * NCH, NCH)]
            pending[bsel] = pltpu.async_copy(ob, dst, osems[bsel])
        for p in pending:
            if p is not None:
                p.wait()

    k(known_feats, idx_b, out_ref)


def kernel(unknown, known, known_feats):
    B, N, _ = unknown.shape
    C = known_feats.shape[1]
    u_t = jnp.transpose(unknown, (0, 2, 1))
    k_t = jnp.transpose(known, (0, 2, 1))
    out_ref = jax.new_ref(jax.lax.empty((B, C, N), jnp.float32))
    for b in range(B):
        idx_b = _tc_argmin_b(u_t, k_t, b)
        _sc_gather_slab(known_feats, idx_b, out_ref, b)
    return out_ref[...]
